# baseline (speedup 1.0000x reference)
import functools

import jax
import jax.numpy as jnp
from jax import lax
from jax.experimental import pallas as pl
from jax.experimental.pallas import tpu as pltpu
from jax.experimental.pallas import tpu_sc as plsc

NUM_EMB = 100000
DIM = 128
B_TOTAL = 4096 * 50
NC, NS = 2, 16
NW = NC * NS
RPC = 128
CPW = B_TOTAL // (NW * RPC)
NBUF = 5
PREF = 3
NORM_BLK = 2000


def _tc_norm_body(w_ref, o_ref):
    w = w_ref[...]
    o_ref[...] = w * lax.rsqrt(
        jnp.mean(w * w, axis=-1, keepdims=True) + 1e-4
    )


_tc_norm = pl.pallas_call(
    _tc_norm_body,
    grid=(NUM_EMB // NORM_BLK,),
    in_specs=[pl.BlockSpec((NORM_BLK, DIM), lambda i: (i, 0))],
    out_specs=pl.BlockSpec((NORM_BLK, DIM), lambda i: (i, 0)),
    out_shape=jax.ShapeDtypeStruct((NUM_EMB, DIM), jnp.float32),
)

_mesh = plsc.VectorSubcoreMesh(core_axis_name="c", subcore_axis_name="s")


@functools.partial(
    pl.kernel,
    mesh=_mesh,
    out_type=jax.ShapeDtypeStruct((B_TOTAL, DIM), jnp.float32),
    scratch_types=[
        pltpu.VMEM((1, CPW, RPC), jnp.int32),
        pltpu.VMEM((NBUF, RPC, DIM), jnp.float32),
        pltpu.SemaphoreType.DMA((NBUF,)),
        pltpu.SemaphoreType.DMA((NBUF,)),
    ],
    compiler_params=pltpu.CompilerParams(needs_layout_passes=False),
)
def _gather(x_hbm, tab_hbm, out_hbm, idx_v, rows_v, gsem, osem):
    wid = lax.axis_index("s") * NC + lax.axis_index("c")
    out_base = wid * CPW * RPC
    pltpu.sync_copy(x_hbm.at[pl.ds(wid, 1)], idx_v)

    def start_gather(ci, b):
        pltpu.async_copy(tab_hbm.at[idx_v.at[0, ci]], rows_v.at[b], gsem.at[b])

    def wait_gather(ci, b):
        pltpu.make_async_copy(
            tab_hbm.at[idx_v.at[0, ci]], rows_v.at[b], gsem.at[b]
        ).wait()

    def out_slice(ci):
        return out_hbm.at[pl.ds(out_base + ci * RPC, RPC)]

    for b in range(PREF):
        start_gather(b, b)

    def outer(o, carry):
        for b in range(NBUF):
            ci = o * NBUF + b
            wait_gather(ci, b)
            pltpu.async_copy(rows_v.at[b], out_slice(ci), osem.at[b])
            cip = ci + PREF
            bp = (b + PREF) % NBUF

            @pl.when(cip < CPW)
            def _():
                @pl.when(cip >= NBUF)
                def _():
                    pltpu.make_async_copy(
                        rows_v.at[bp], out_slice(cip), osem.at[bp]
                    ).wait()

                start_gather(cip, bp)

        return carry

    lax.fori_loop(0, CPW // NBUF, outer, 0)
    for b in range(NBUF):
        pltpu.make_async_copy(rows_v.at[b], out_slice(b), osem.at[b]).wait()


def kernel(x, weight):
    normed = _tc_norm(weight)
    x2 = x.astype(jnp.int32).reshape(NW, CPW, RPC)
    out = _gather(x2, normed)
    return out.reshape(4096, 50, DIM)
